# baseline (speedup 1.0000x reference)
import functools

import jax
import jax.numpy as jnp
from jax import lax
from jax.experimental import pallas as pl
from jax.experimental.pallas import tpu as pltpu
from jax.experimental.pallas import tpu_sc as plsc

_BATCH = 16384
_CHUNK = 128
_NW = 32
_CPW = _BATCH // _CHUNK // _NW


def _sc_gather(uidx2, iidx2, U_gmf, I_gmf, U_mlp, I_mlp):
    nrow = _BATCH // _CHUNK
    mesh = plsc.VectorSubcoreMesh(core_axis_name="c", subcore_axis_name="s")

    @functools.partial(
        pl.kernel,
        mesh=mesh,
        compiler_params=pltpu.CompilerParams(use_tc_tiling_on_sc=False),
        out_type=[
            jax.ShapeDtypeStruct((nrow, _CHUNK, 8), jnp.float32),
            jax.ShapeDtypeStruct((nrow, _CHUNK, 8), jnp.float32),
            jax.ShapeDtypeStruct((nrow, _CHUNK, 32), jnp.float32),
            jax.ShapeDtypeStruct((nrow, _CHUNK, 32), jnp.float32),
        ],
        scratch_types=[
            pltpu.VMEM((_CPW, _CHUNK), jnp.int32),
            pltpu.VMEM((_CPW, _CHUNK), jnp.int32),
            pltpu.VMEM((_CPW, _CHUNK, 8), jnp.float32),
            pltpu.VMEM((_CPW, _CHUNK, 8), jnp.float32),
            pltpu.VMEM((_CPW, _CHUNK, 32), jnp.float32),
            pltpu.VMEM((_CPW, _CHUNK, 32), jnp.float32),
            pltpu.SemaphoreType.DMA,
        ],
    )
    def k(uidx_hbm, iidx_hbm, ug_t, ig_t, um_t, im_t,
          ug_o, ig_o, um_o, im_o,
          uv, iv, ugv, igv, umv, imv, sem):
        wid = lax.axis_index("s") * 2 + lax.axis_index("c")
        base = wid * _CPW
        pltpu.sync_copy(uidx_hbm.at[pl.ds(base, _CPW)], uv)
        pltpu.sync_copy(iidx_hbm.at[pl.ds(base, _CPW)], iv)
        copies = []
        for j in range(_CPW):
            copies.append(pltpu.async_copy(ug_t.at[uv.at[j]], ugv.at[j], sem))
            copies.append(pltpu.async_copy(ig_t.at[iv.at[j]], igv.at[j], sem))
            copies.append(pltpu.async_copy(um_t.at[uv.at[j]], umv.at[j], sem))
            copies.append(pltpu.async_copy(im_t.at[iv.at[j]], imv.at[j], sem))
        for c in copies:
            c.wait()
        pltpu.sync_copy(ugv, ug_o.at[pl.ds(base, _CPW)])
        pltpu.sync_copy(igv, ig_o.at[pl.ds(base, _CPW)])
        pltpu.sync_copy(umv, um_o.at[pl.ds(base, _CPW)])
        pltpu.sync_copy(imv, im_o.at[pl.ds(base, _CPW)])

    return k(uidx2, iidx2, U_gmf, I_gmf, U_mlp, I_mlp)


def _tc_head(ug, ig, um, im, W1u, W1i, b1, W2, b2, W3, b3, Wfg, Wfh, bf):
    bb = 2048
    grid = (_BATCH // bb,)

    def body(ug_r, ig_r, um_r, im_r, w1u_r, w1i_r, b1_r, w2_r, b2_r,
             w3_r, b3_r, wfg_r, wfh_r, bf_r, o_r):
        g = ug_r[...] * ig_r[...]
        h = jnp.dot(um_r[...], w1u_r[...], preferred_element_type=jnp.float32)
        h = h + jnp.dot(im_r[...], w1i_r[...], preferred_element_type=jnp.float32)
        h = jnp.maximum(h + b1_r[...], 0.0)
        h = jnp.maximum(
            jnp.dot(h, w2_r[...], preferred_element_type=jnp.float32) + b2_r[...], 0.0)
        h = jnp.maximum(
            jnp.dot(h, w3_r[...], preferred_element_type=jnp.float32) + b3_r[...], 0.0)
        s = jnp.dot(g, wfg_r[...], preferred_element_type=jnp.float32)
        s = s + jnp.dot(h, wfh_r[...], preferred_element_type=jnp.float32)
        s = s + bf_r[...]
        o_r[...] = jax.nn.sigmoid(s)

    batch_spec = lambda d: pl.BlockSpec((bb, d), lambda i: (i, 0))
    full_spec = lambda a: pl.BlockSpec(a.shape, lambda i: (0,) * a.ndim)
    return pl.pallas_call(
        body,
        grid=grid,
        in_specs=[
            batch_spec(8), batch_spec(8), batch_spec(32), batch_spec(32),
            full_spec(W1u), full_spec(W1i), full_spec(b1),
            full_spec(W2), full_spec(b2), full_spec(W3), full_spec(b3),
            full_spec(Wfg), full_spec(Wfh), full_spec(bf),
        ],
        out_specs=pl.BlockSpec((bb, 1), lambda i: (i, 0)),
        out_shape=jax.ShapeDtypeStruct((_BATCH, 1), jnp.float32),
    )(ug, ig, um, im, W1u, W1i, b1, W2, b2, W3, b3, Wfg, Wfh, bf)


def kernel(user_indices, item_indices, U_gmf, I_gmf, U_mlp, I_mlp,
           W1, b1, W2, b2, W3, b3, Wf, bf):
    uidx2 = user_indices.astype(jnp.int32).reshape(_BATCH // _CHUNK, _CHUNK)
    iidx2 = item_indices.astype(jnp.int32).reshape(_BATCH // _CHUNK, _CHUNK)
    ug, ig, um, im = _sc_gather(uidx2, iidx2, U_gmf, I_gmf, U_mlp, I_mlp)
    ug = ug.reshape(_BATCH, 8)
    ig = ig.reshape(_BATCH, 8)
    um = um.reshape(_BATCH, 32)
    im = im.reshape(_BATCH, 32)
    return _tc_head(
        ug, ig, um, im,
        W1[:32], W1[32:], b1.reshape(1, 32),
        W2, b2.reshape(1, 16), W3, b3.reshape(1, 8),
        Wf[:8], Wf[8:], bf.reshape(1, 1),
    )


# speedup vs baseline: 2.7685x; 2.7685x over previous
import functools

import jax
import jax.numpy as jnp
from jax import lax
from jax.experimental import pallas as pl
from jax.experimental.pallas import tpu as pltpu
from jax.experimental.pallas import tpu_sc as plsc

_BATCH = 16384
_NW = 32
_BPW = _BATCH // _NW
_G = 8


def _sc_gather_t(uidx, iidx, ug_t, ig_t, um_t, im_t):
    mesh = plsc.VectorSubcoreMesh(core_axis_name="c", subcore_axis_name="s")

    @functools.partial(
        pl.kernel,
        mesh=mesh,
        compiler_params=pltpu.CompilerParams(
            use_tc_tiling_on_sc=True, needs_layout_passes=False),
        out_type=[
            jax.ShapeDtypeStruct((8, _BATCH), jnp.float32),
            jax.ShapeDtypeStruct((8, _BATCH), jnp.float32),
            jax.ShapeDtypeStruct((32, _BATCH), jnp.float32),
            jax.ShapeDtypeStruct((32, _BATCH), jnp.float32),
        ],
        scratch_types=[
            pltpu.VMEM((_BPW + 16,), jnp.int32),
            pltpu.VMEM((_BPW + 16,), jnp.int32),
            pltpu.VMEM((_G, 8, 128), jnp.float32),
            pltpu.VMEM((_G, 8, 128), jnp.float32),
            pltpu.VMEM((_G, 32, 128), jnp.float32),
            pltpu.VMEM((_G, 32, 128), jnp.float32),
            pltpu.VMEM((8, _BPW), jnp.float32),
            pltpu.VMEM((8, _BPW), jnp.float32),
            pltpu.VMEM((32, _BPW), jnp.float32),
            pltpu.VMEM((32, _BPW), jnp.float32),
            pltpu.SemaphoreType.DMA,
        ],
    )
    def k(uidx_hbm, iidx_hbm, ugt, igt, umt, imt,
          ug_o, ig_o, um_o, im_o,
          us, is_, gbu, gbi, mbu, mbi, ugv, igv, umv, imv, sem):
        wid = lax.axis_index("s") * 2 + lax.axis_index("c")
        base = wid * _BPW
        pltpu.sync_copy(uidx_hbm.at[pl.ds(base, _BPW)], us.at[pl.ds(0, _BPW)])
        pltpu.sync_copy(iidx_hbm.at[pl.ds(base, _BPW)], is_.at[pl.ds(0, _BPW)])

        rows16 = lax.iota(jnp.int32, 16)
        m8 = rows16 < 8

        def group(g, carry):
            uvec = us[pl.ds(g * _G, 16)]
            ivec = is_[pl.ds(g * _G, 16)]
            for e in range(_G):
                r = uvec[e]
                q = ivec[e]
                ju = pl.multiple_of((r >> 7) << 7, 128)
                ji = pl.multiple_of((q >> 7) << 7, 128)
                pltpu.make_async_copy(
                    ugt.at[:, pl.ds(ju, 128)], gbu.at[e], sem).start()
                pltpu.make_async_copy(
                    igt.at[:, pl.ds(ji, 128)], gbi.at[e], sem).start()
                pltpu.make_async_copy(
                    umt.at[:, pl.ds(ju, 128)], mbu.at[e], sem).start()
                pltpu.make_async_copy(
                    imt.at[:, pl.ds(ji, 128)], mbi.at[e], sem).start()
            for e in range(_G):
                pltpu.make_async_copy(
                    ugt.at[:, pl.ds(0, 128)], gbu.at[e], sem).wait()
                pltpu.make_async_copy(
                    igt.at[:, pl.ds(0, 128)], gbi.at[e], sem).wait()
                pltpu.make_async_copy(
                    umt.at[:, pl.ds(0, 128)], mbu.at[e], sem).wait()
                pltpu.make_async_copy(
                    imt.at[:, pl.ds(0, 128)], mbi.at[e], sem).wait()
            lvu = uvec & 127
            lvi = ivec & 127
            for e in range(_G):
                kk = g * _G + e
                k16 = jnp.full((16,), kk, jnp.int32)
                lu = jnp.full((16,), lvu[e], jnp.int32)
                li = jnp.full((16,), lvi[e], jnp.int32)
                vg = plsc.load_gather(gbu.at[e], [rows16, lu], mask=m8)
                plsc.store_scatter(ugv, [rows16, k16], vg, mask=m8)
                vg = plsc.load_gather(gbi.at[e], [rows16, li], mask=m8)
                plsc.store_scatter(igv, [rows16, k16], vg, mask=m8)
                for h in range(2):
                    rh = rows16 + (16 * h)
                    vm = plsc.load_gather(mbu.at[e], [rh, lu])
                    plsc.store_scatter(umv, [rh, k16], vm)
                    vm = plsc.load_gather(mbi.at[e], [rh, li])
                    plsc.store_scatter(imv, [rh, k16], vm)
            return carry

        lax.fori_loop(0, _BPW // _G, group, 0)

        pltpu.sync_copy(ugv, ug_o.at[:, pl.ds(base, _BPW)])
        pltpu.sync_copy(igv, ig_o.at[:, pl.ds(base, _BPW)])
        pltpu.sync_copy(umv, um_o.at[:, pl.ds(base, _BPW)])
        pltpu.sync_copy(imv, im_o.at[:, pl.ds(base, _BPW)])

    return k(uidx, iidx, ug_t, ig_t, um_t, im_t)


def _tc_head_t(ugT, igT, umT, imT, W1uT, W1iT, b1c, W2T, b2c, W3T, b3c,
               Wfg, Wfh, bf):
    bb = 2048
    grid = (_BATCH // bb,)

    def body(ug_r, ig_r, um_r, im_r, w1u_r, w1i_r, b1_r, w2_r, b2_r,
             w3_r, b3_r, wfg_r, wfh_r, bf_r, o_r):
        g = ug_r[...] * ig_r[...]
        h = jnp.dot(w1u_r[...], um_r[...], preferred_element_type=jnp.float32)
        h = h + jnp.dot(w1i_r[...], im_r[...], preferred_element_type=jnp.float32)
        h = jnp.maximum(h + b1_r[...], 0.0)
        h = jnp.maximum(
            jnp.dot(w2_r[...], h, preferred_element_type=jnp.float32)
            + b2_r[...], 0.0)
        h = jnp.maximum(
            jnp.dot(w3_r[...], h, preferred_element_type=jnp.float32)
            + b3_r[...], 0.0)
        dn = (((0,), (0,)), ((), ()))
        s = lax.dot_general(g, wfg_r[...], dn,
                            preferred_element_type=jnp.float32)
        s = s + lax.dot_general(h, wfh_r[...], dn,
                                preferred_element_type=jnp.float32)
        s = s + bf_r[...]
        o_r[...] = jax.nn.sigmoid(s)

    batch_spec = lambda d: pl.BlockSpec((d, bb), lambda i: (0, i))
    full_spec = lambda a: pl.BlockSpec(a.shape, lambda i: (0,) * a.ndim)
    return pl.pallas_call(
        body,
        grid=grid,
        in_specs=[
            batch_spec(8), batch_spec(8), batch_spec(32), batch_spec(32),
            full_spec(W1uT), full_spec(W1iT), full_spec(b1c),
            full_spec(W2T), full_spec(b2c), full_spec(W3T), full_spec(b3c),
            full_spec(Wfg), full_spec(Wfh), full_spec(bf),
        ],
        out_specs=pl.BlockSpec((bb, 1), lambda i: (i, 0)),
        out_shape=jax.ShapeDtypeStruct((_BATCH, 1), jnp.float32),
    )(ugT, igT, umT, imT, W1uT, W1iT, b1c, W2T, b2c, W3T, b3c, Wfg, Wfh, bf)


def kernel(user_indices, item_indices, U_gmf, I_gmf, U_mlp, I_mlp,
           W1, b1, W2, b2, W3, b3, Wf, bf):
    uidx = user_indices.astype(jnp.int32)
    iidx = item_indices.astype(jnp.int32)
    ugT, igT, umT, imT = _sc_gather_t(
        uidx, iidx, U_gmf.T, I_gmf.T, U_mlp.T, I_mlp.T)
    return _tc_head_t(
        ugT, igT, umT, imT,
        W1[:32].T, W1[32:].T, b1.reshape(32, 1),
        W2.T, b2.reshape(16, 1), W3.T, b3.reshape(8, 1),
        Wf[:8], Wf[8:], bf.reshape(1, 1),
    )


# speedup vs baseline: 4.0929x; 1.4784x over previous
import functools

import jax
import jax.numpy as jnp
from jax import lax
from jax.experimental import pallas as pl
from jax.experimental.pallas import tpu as pltpu
from jax.experimental.pallas import tpu_sc as plsc

_BATCH = 16384
_NW = 32
_BPW = _BATCH // _NW
_NU = 1000000
_NI = 100000
_CUT_U = (_NU // 128) * 128
_CUT_I = (_NI // 128) * 128
_WP_U = _CUT_U * 8
_WP_I = _CUT_I * 8


def _flat_view(tab, vcut):
    d = tab.shape[1]
    p = tab.T[:, :vcut]
    b = p.reshape(d // 8, 8, vcut // 128, 128)
    return b.transpose(0, 2, 1, 3).reshape(-1)


def _sc_gather_t(uidx, iidx, fug, fig, fum, fim, ugt, igt, umt, imt):
    mesh = plsc.VectorSubcoreMesh(core_axis_name="c", subcore_axis_name="s")

    @functools.partial(
        pl.kernel,
        mesh=mesh,
        compiler_params=pltpu.CompilerParams(
            use_tc_tiling_on_sc=True, needs_layout_passes=False),
        out_type=[
            jax.ShapeDtypeStruct((8, _BATCH), jnp.float32),
            jax.ShapeDtypeStruct((8, _BATCH), jnp.float32),
            jax.ShapeDtypeStruct((32, _BATCH), jnp.float32),
            jax.ShapeDtypeStruct((32, _BATCH), jnp.float32),
        ],
        scratch_types=[
            pltpu.VMEM((_BPW + 16,), jnp.int32),
            pltpu.VMEM((_BPW + 16,), jnp.int32),
            pltpu.VMEM((40, 128), jnp.int32),
            pltpu.VMEM((40, 128), jnp.int32),
            pltpu.VMEM((8, 128), jnp.float32),
            pltpu.VMEM((32, 128), jnp.float32),
            pltpu.VMEM((8, _BPW), jnp.float32),
            pltpu.VMEM((8, _BPW), jnp.float32),
            pltpu.VMEM((32, _BPW), jnp.float32),
            pltpu.VMEM((32, _BPW), jnp.float32),
            pltpu.SemaphoreType.DMA,
        ],
    )
    def k(uidx_hbm, iidx_hbm, fug_r, fig_r, fum_r, fim_r,
          ugt_r, igt_r, umt_r, imt_r,
          ug_o, ig_o, um_o, im_o,
          us, is_, ibu, ibi, tgb, tmb, ugv, igv, umv, imv, sem):
        wid = lax.axis_index("s") * 2 + lax.axis_index("c")
        base = wid * _BPW
        pltpu.sync_copy(uidx_hbm.at[pl.ds(base, _BPW)], us.at[pl.ds(0, _BPW)])
        pltpu.sync_copy(iidx_hbm.at[pl.ds(base, _BPW)], is_.at[pl.ds(0, _BPW)])

        rows16 = lax.iota(jnp.int32, 16)
        m8 = rows16 < 8

        def chunk(ch, carry):
            cb = ch * 128
            for t in range(8):
                uv = jnp.minimum(us[pl.ds(cb + t * 16, 16)], _CUT_U - 1)
                qv = jnp.minimum(is_[pl.ds(cb + t * 16, 16)], _CUT_I - 1)
                ju = (uv >> 7) * 1024 + (uv & 127)
                ji = (qv >> 7) * 1024 + (qv & 127)
                for c in range(8):
                    ibu[c, pl.ds(t * 16, 16)] = ju + c * 128
                    ibi[c, pl.ds(t * 16, 16)] = ji + c * 128
                for c in range(32):
                    ibu[8 + c, pl.ds(t * 16, 16)] = (
                        ju + (c // 8) * _WP_U + (c % 8) * 128)
                    ibi[8 + c, pl.ds(t * 16, 16)] = (
                        ji + (c // 8) * _WP_I + (c % 8) * 128)
            for c in range(8):
                pltpu.make_async_copy(
                    fug_r.at[ibu.at[c]], ugv.at[c, pl.ds(cb, 128)], sem).start()
                pltpu.make_async_copy(
                    fig_r.at[ibi.at[c]], igv.at[c, pl.ds(cb, 128)], sem).start()
            for c in range(32):
                pltpu.make_async_copy(
                    fum_r.at[ibu.at[8 + c]], umv.at[c, pl.ds(cb, 128)],
                    sem).start()
                pltpu.make_async_copy(
                    fim_r.at[ibi.at[8 + c]], imv.at[c, pl.ds(cb, 128)],
                    sem).start()
            for c in range(8):
                pltpu.make_async_copy(
                    fug_r.at[ibu.at[c]], ugv.at[c, pl.ds(cb, 128)], sem).wait()
                pltpu.make_async_copy(
                    fig_r.at[ibi.at[c]], igv.at[c, pl.ds(cb, 128)], sem).wait()
            for c in range(32):
                pltpu.make_async_copy(
                    fum_r.at[ibu.at[8 + c]], umv.at[c, pl.ds(cb, 128)],
                    sem).wait()
                pltpu.make_async_copy(
                    fim_r.at[ibi.at[8 + c]], imv.at[c, pl.ds(cb, 128)],
                    sem).wait()
            return carry

        lax.fori_loop(0, _BPW // 128, chunk, 0)

        def fixup(kk, carry):
            r = us[pl.ds(kk, 16)][0]
            q = is_[pl.ds(kk, 16)][0]
            k16 = jnp.full((16,), kk, jnp.int32)

            @pl.when(r >= _CUT_U)
            def _():
                ju = pl.multiple_of((r >> 7) << 7, 128)
                pltpu.sync_copy(ugt_r.at[:, pl.ds(ju, 128)], tgb)
                pltpu.sync_copy(umt_r.at[:, pl.ds(ju, 128)], tmb)
                l16 = jnp.full((16,), r & 127, jnp.int32)
                vg = plsc.load_gather(tgb, [rows16, l16], mask=m8)
                plsc.store_scatter(ugv, [rows16, k16], vg, mask=m8)
                for h in range(2):
                    rh = rows16 + (16 * h)
                    vm = plsc.load_gather(tmb, [rh, l16])
                    plsc.store_scatter(umv, [rh, k16], vm)

            @pl.when(q >= _CUT_I)
            def _():
                ji = pl.multiple_of((q >> 7) << 7, 128)
                pltpu.sync_copy(igt_r.at[:, pl.ds(ji, 128)], tgb)
                pltpu.sync_copy(imt_r.at[:, pl.ds(ji, 128)], tmb)
                l16 = jnp.full((16,), q & 127, jnp.int32)
                vg = plsc.load_gather(tgb, [rows16, l16], mask=m8)
                plsc.store_scatter(igv, [rows16, k16], vg, mask=m8)
                for h in range(2):
                    rh = rows16 + (16 * h)
                    vm = plsc.load_gather(tmb, [rh, l16])
                    plsc.store_scatter(imv, [rh, k16], vm)
            return carry

        lax.fori_loop(0, _BPW, fixup, 0)

        pltpu.sync_copy(ugv, ug_o.at[:, pl.ds(base, _BPW)])
        pltpu.sync_copy(igv, ig_o.at[:, pl.ds(base, _BPW)])
        pltpu.sync_copy(umv, um_o.at[:, pl.ds(base, _BPW)])
        pltpu.sync_copy(imv, im_o.at[:, pl.ds(base, _BPW)])

    return k(uidx, iidx, fug, fig, fum, fim, ugt, igt, umt, imt)


def _tc_head_t(ugT, igT, umT, imT, W1uT, W1iT, b1c, W2T, b2c, W3T, b3c,
               Wfg, Wfh, bf):
    bb = 2048
    grid = (_BATCH // bb,)

    def body(ug_r, ig_r, um_r, im_r, w1u_r, w1i_r, b1_r, w2_r, b2_r,
             w3_r, b3_r, wfg_r, wfh_r, bf_r, o_r):
        g = ug_r[...] * ig_r[...]
        h = jnp.dot(w1u_r[...], um_r[...], preferred_element_type=jnp.float32)
        h = h + jnp.dot(w1i_r[...], im_r[...], preferred_element_type=jnp.float32)
        h = jnp.maximum(h + b1_r[...], 0.0)
        h = jnp.maximum(
            jnp.dot(w2_r[...], h, preferred_element_type=jnp.float32)
            + b2_r[...], 0.0)
        h = jnp.maximum(
            jnp.dot(w3_r[...], h, preferred_element_type=jnp.float32)
            + b3_r[...], 0.0)
        dn = (((0,), (0,)), ((), ()))
        s = lax.dot_general(g, wfg_r[...], dn,
                            preferred_element_type=jnp.float32)
        s = s + lax.dot_general(h, wfh_r[...], dn,
                                preferred_element_type=jnp.float32)
        s = s + bf_r[...]
        o_r[...] = jax.nn.sigmoid(s)

    batch_spec = lambda d: pl.BlockSpec((d, bb), lambda i: (0, i))
    full_spec = lambda a: pl.BlockSpec(a.shape, lambda i: (0,) * a.ndim)
    return pl.pallas_call(
        body,
        grid=grid,
        in_specs=[
            batch_spec(8), batch_spec(8), batch_spec(32), batch_spec(32),
            full_spec(W1uT), full_spec(W1iT), full_spec(b1c),
            full_spec(W2T), full_spec(b2c), full_spec(W3T), full_spec(b3c),
            full_spec(Wfg), full_spec(Wfh), full_spec(bf),
        ],
        out_specs=pl.BlockSpec((bb, 1), lambda i: (i, 0)),
        out_shape=jax.ShapeDtypeStruct((_BATCH, 1), jnp.float32),
    )(ugT, igT, umT, imT, W1uT, W1iT, b1c, W2T, b2c, W3T, b3c, Wfg, Wfh, bf)


def kernel(user_indices, item_indices, U_gmf, I_gmf, U_mlp, I_mlp,
           W1, b1, W2, b2, W3, b3, Wf, bf):
    uidx = user_indices.astype(jnp.int32)
    iidx = item_indices.astype(jnp.int32)
    ugT, igT, umT, imT = _sc_gather_t(
        uidx, iidx,
        _flat_view(U_gmf, _CUT_U), _flat_view(I_gmf, _CUT_I),
        _flat_view(U_mlp, _CUT_U), _flat_view(I_mlp, _CUT_I),
        U_gmf.T, I_gmf.T, U_mlp.T, I_mlp.T)
    return _tc_head_t(
        ugT, igT, umT, imT,
        W1[:32].T, W1[32:].T, b1.reshape(32, 1),
        W2.T, b2.reshape(16, 1), W3.T, b3.reshape(8, 1),
        Wf[:8], Wf[8:], bf.reshape(1, 1),
    )


# speedup vs baseline: 4.5331x; 1.1075x over previous
import functools

import jax
import jax.numpy as jnp
from jax import lax
from jax.experimental import pallas as pl
from jax.experimental.pallas import tpu as pltpu
from jax.experimental.pallas import tpu_sc as plsc

_BATCH = 16384
_NW = 32
_BPW = _BATCH // _NW
_NU = 1000000
_NI = 100000
_CUT_U = (_NU // 128) * 128
_CUT_I = (_NI // 128) * 128
_WP_U = _CUT_U * 8
_WP_I = _CUT_I * 8


def _flat_view(tab, vcut):
    d = tab.shape[1]
    p = tab.T[:, :vcut]
    b = p.reshape(d // 8, 8, vcut // 128, 128)
    return b.transpose(0, 2, 1, 3).reshape(-1)


def _sc_gather_t(uidx, iidx, fug, fig, fum, fim, ugt, igt, umt, imt):
    mesh = plsc.VectorSubcoreMesh(core_axis_name="c", subcore_axis_name="s")

    @functools.partial(
        pl.kernel,
        mesh=mesh,
        compiler_params=pltpu.CompilerParams(
            use_tc_tiling_on_sc=True, needs_layout_passes=False),
        out_type=[
            jax.ShapeDtypeStruct((8, _BATCH), jnp.float32),
            jax.ShapeDtypeStruct((8, _BATCH), jnp.float32),
            jax.ShapeDtypeStruct((32, _BATCH), jnp.float32),
            jax.ShapeDtypeStruct((32, _BATCH), jnp.float32),
        ],
        scratch_types=[
            pltpu.VMEM((_BPW + 16,), jnp.int32),
            pltpu.VMEM((_BPW + 16,), jnp.int32),
            pltpu.VMEM((40, 128), jnp.int32),
            pltpu.VMEM((40, 128), jnp.int32),
            pltpu.VMEM((40, 128), jnp.int32),
            pltpu.VMEM((40, 128), jnp.int32),
            pltpu.VMEM((8, 128), jnp.float32),
            pltpu.VMEM((32, 128), jnp.float32),
            pltpu.VMEM((8, _BPW), jnp.float32),
            pltpu.VMEM((8, _BPW), jnp.float32),
            pltpu.VMEM((32, _BPW), jnp.float32),
            pltpu.VMEM((32, _BPW), jnp.float32),
            pltpu.SemaphoreType.DMA,
        ],
    )
    def k(uidx_hbm, iidx_hbm, fug_r, fig_r, fum_r, fim_r,
          ugt_r, igt_r, umt_r, imt_r,
          ug_o, ig_o, um_o, im_o,
          us, is_, ibu, ibi, ibu2, ibi2, tgb, tmb, ugv, igv, umv, imv, sem):
        wid = lax.axis_index("s") * 2 + lax.axis_index("c")
        base = wid * _BPW
        pltpu.sync_copy(uidx_hbm.at[pl.ds(base, _BPW)], us.at[pl.ds(0, _BPW)])
        pltpu.sync_copy(iidx_hbm.at[pl.ds(base, _BPW)], is_.at[pl.ds(0, _BPW)])

        rows16 = lax.iota(jnp.int32, 16)
        m8 = rows16 < 8

        def build(cb, bu, bi):
            for t in range(8):
                uv = jnp.minimum(us[pl.ds(cb + t * 16, 16)], _CUT_U - 1)
                qv = jnp.minimum(is_[pl.ds(cb + t * 16, 16)], _CUT_I - 1)
                ju = (uv >> 7) * 1024 + (uv & 127)
                ji = (qv >> 7) * 1024 + (qv & 127)
                for c in range(8):
                    bu[c, pl.ds(t * 16, 16)] = ju + c * 128
                    bi[c, pl.ds(t * 16, 16)] = ji + c * 128
                for c in range(32):
                    bu[8 + c, pl.ds(t * 16, 16)] = (
                        ju + (c // 8) * _WP_U + (c % 8) * 128)
                    bi[8 + c, pl.ds(t * 16, 16)] = (
                        ji + (c // 8) * _WP_I + (c % 8) * 128)

        def copies(cb, bu, bi):
            out = []
            for c in range(8):
                out.append(pltpu.make_async_copy(
                    fug_r.at[bu.at[c]], ugv.at[c, pl.ds(cb, 128)], sem))
                out.append(pltpu.make_async_copy(
                    fig_r.at[bi.at[c]], igv.at[c, pl.ds(cb, 128)], sem))
            for c in range(32):
                out.append(pltpu.make_async_copy(
                    fum_r.at[bu.at[8 + c]], umv.at[c, pl.ds(cb, 128)], sem))
                out.append(pltpu.make_async_copy(
                    fim_r.at[bi.at[8 + c]], imv.at[c, pl.ds(cb, 128)], sem))
            return out

        def superchunk(g, carry):
            c0 = g * 256
            c1 = c0 + 128
            build(c0, ibu, ibi)
            for cp in copies(c0, ibu, ibi):
                cp.start()
            build(c1, ibu2, ibi2)
            for cp in copies(c1, ibu2, ibi2):
                cp.start()
            for cp in copies(c0, ibu, ibi):
                cp.wait()
            for cp in copies(c1, ibu2, ibi2):
                cp.wait()
            return carry

        lax.fori_loop(0, _BPW // 256, superchunk, 0)

        def fixup(gg, carry):
            gb = gg * 16
            uv = us[pl.ds(gb, 16)]
            qv = is_[pl.ds(gb, 16)]

            @pl.when(jnp.max(uv, axis=0) >= _CUT_U)
            def _():
                for e in range(16):
                    r = uv[e]

                    @pl.when(r >= _CUT_U)
                    def _():
                        ju = pl.multiple_of((r >> 7) << 7, 128)
                        pltpu.sync_copy(ugt_r.at[:, pl.ds(ju, 128)], tgb)
                        pltpu.sync_copy(umt_r.at[:, pl.ds(ju, 128)], tmb)
                        k16 = jnp.full((16,), gb + e, jnp.int32)
                        l16 = jnp.full((16,), r & 127, jnp.int32)
                        vg = plsc.load_gather(tgb, [rows16, l16], mask=m8)
                        plsc.store_scatter(ugv, [rows16, k16], vg, mask=m8)
                        for h in range(2):
                            rh = rows16 + (16 * h)
                            vm = plsc.load_gather(tmb, [rh, l16])
                            plsc.store_scatter(umv, [rh, k16], vm)

            @pl.when(jnp.max(qv, axis=0) >= _CUT_I)
            def _():
                for e in range(16):
                    q = qv[e]

                    @pl.when(q >= _CUT_I)
                    def _():
                        ji = pl.multiple_of((q >> 7) << 7, 128)
                        pltpu.sync_copy(igt_r.at[:, pl.ds(ji, 128)], tgb)
                        pltpu.sync_copy(imt_r.at[:, pl.ds(ji, 128)], tmb)
                        k16 = jnp.full((16,), gb + e, jnp.int32)
                        l16 = jnp.full((16,), q & 127, jnp.int32)
                        vg = plsc.load_gather(tgb, [rows16, l16], mask=m8)
                        plsc.store_scatter(igv, [rows16, k16], vg, mask=m8)
                        for h in range(2):
                            rh = rows16 + (16 * h)
                            vm = plsc.load_gather(tmb, [rh, l16])
                            plsc.store_scatter(imv, [rh, k16], vm)
            return carry

        lax.fori_loop(0, _BPW // 16, fixup, 0)

        pltpu.sync_copy(ugv, ug_o.at[:, pl.ds(base, _BPW)])
        pltpu.sync_copy(igv, ig_o.at[:, pl.ds(base, _BPW)])
        pltpu.sync_copy(umv, um_o.at[:, pl.ds(base, _BPW)])
        pltpu.sync_copy(imv, im_o.at[:, pl.ds(base, _BPW)])

    return k(uidx, iidx, fug, fig, fum, fim, ugt, igt, umt, imt)


def _tc_head_t(ugT, igT, umT, imT, W1uT, W1iT, b1c, W2T, b2c, W3T, b3c,
               Wfg, Wfh, bf):
    bb = 2048
    grid = (_BATCH // bb,)

    def body(ug_r, ig_r, um_r, im_r, w1u_r, w1i_r, b1_r, w2_r, b2_r,
             w3_r, b3_r, wfg_r, wfh_r, bf_r, o_r):
        g = ug_r[...] * ig_r[...]
        h = jnp.dot(w1u_r[...], um_r[...], preferred_element_type=jnp.float32)
        h = h + jnp.dot(w1i_r[...], im_r[...], preferred_element_type=jnp.float32)
        h = jnp.maximum(h + b1_r[...], 0.0)
        h = jnp.maximum(
            jnp.dot(w2_r[...], h, preferred_element_type=jnp.float32)
            + b2_r[...], 0.0)
        h = jnp.maximum(
            jnp.dot(w3_r[...], h, preferred_element_type=jnp.float32)
            + b3_r[...], 0.0)
        dn = (((0,), (0,)), ((), ()))
        s = lax.dot_general(g, wfg_r[...], dn,
                            preferred_element_type=jnp.float32)
        s = s + lax.dot_general(h, wfh_r[...], dn,
                                preferred_element_type=jnp.float32)
        s = s + bf_r[...]
        o_r[...] = jax.nn.sigmoid(s)

    batch_spec = lambda d: pl.BlockSpec((d, bb), lambda i: (0, i))
    full_spec = lambda a: pl.BlockSpec(a.shape, lambda i: (0,) * a.ndim)
    return pl.pallas_call(
        body,
        grid=grid,
        in_specs=[
            batch_spec(8), batch_spec(8), batch_spec(32), batch_spec(32),
            full_spec(W1uT), full_spec(W1iT), full_spec(b1c),
            full_spec(W2T), full_spec(b2c), full_spec(W3T), full_spec(b3c),
            full_spec(Wfg), full_spec(Wfh), full_spec(bf),
        ],
        out_specs=pl.BlockSpec((bb, 1), lambda i: (i, 0)),
        out_shape=jax.ShapeDtypeStruct((_BATCH, 1), jnp.float32),
    )(ugT, igT, umT, imT, W1uT, W1iT, b1c, W2T, b2c, W3T, b3c, Wfg, Wfh, bf)


def kernel(user_indices, item_indices, U_gmf, I_gmf, U_mlp, I_mlp,
           W1, b1, W2, b2, W3, b3, Wf, bf):
    uidx = user_indices.astype(jnp.int32)
    iidx = item_indices.astype(jnp.int32)
    ugT, igT, umT, imT = _sc_gather_t(
        uidx, iidx,
        _flat_view(U_gmf, _CUT_U), _flat_view(I_gmf, _CUT_I),
        _flat_view(U_mlp, _CUT_U), _flat_view(I_mlp, _CUT_I),
        U_gmf.T, I_gmf.T, U_mlp.T, I_mlp.T)
    return _tc_head_t(
        ugT, igT, umT, imT,
        W1[:32].T, W1[32:].T, b1.reshape(32, 1),
        W2.T, b2.reshape(16, 1), W3.T, b3.reshape(8, 1),
        Wf[:8], Wf[8:], bf.reshape(1, 1),
    )


# speedup vs baseline: 4.5446x; 1.0025x over previous
import functools

import jax
import jax.numpy as jnp
from jax import lax
from jax.experimental import pallas as pl
from jax.experimental.pallas import tpu as pltpu
from jax.experimental.pallas import tpu_sc as plsc

_BATCH = 16384
_NW = 32
_BPW = _BATCH // _NW
_NU = 1000000
_NI = 100000
_CUT_U = (_NU // 128) * 128
_CUT_I = (_NI // 128) * 128
_WP_U = _CUT_U * 8
_WP_I = _CUT_I * 8


def _flat_view(tab, vcut):
    d = tab.shape[1]
    p = tab.T[:, :vcut]
    b = p.reshape(d // 8, 8, vcut // 128, 128)
    return b.transpose(0, 2, 1, 3).reshape(-1)


def _sc_gather_t(uidx, iidx, fug, fig, fum, fim, ugt, igt, umt, imt):
    mesh = plsc.VectorSubcoreMesh(core_axis_name="c", subcore_axis_name="s")

    @functools.partial(
        pl.kernel,
        mesh=mesh,
        compiler_params=pltpu.CompilerParams(
            use_tc_tiling_on_sc=True, needs_layout_passes=False),
        out_type=[
            jax.ShapeDtypeStruct((8, _BATCH), jnp.float32),
            jax.ShapeDtypeStruct((8, _BATCH), jnp.float32),
            jax.ShapeDtypeStruct((32, _BATCH), jnp.float32),
            jax.ShapeDtypeStruct((32, _BATCH), jnp.float32),
        ],
        scratch_types=[
            pltpu.VMEM((_BPW + 16,), jnp.int32),
            pltpu.VMEM((_BPW + 16,), jnp.int32),
            pltpu.VMEM((40, 128), jnp.int32),
            pltpu.VMEM((40, 128), jnp.int32),
            pltpu.VMEM((40, 128), jnp.int32),
            pltpu.VMEM((40, 128), jnp.int32),
            pltpu.VMEM((8, 128), jnp.float32),
            pltpu.VMEM((32, 128), jnp.float32),
            pltpu.VMEM((8, _BPW), jnp.float32),
            pltpu.VMEM((8, _BPW), jnp.float32),
            pltpu.VMEM((32, _BPW), jnp.float32),
            pltpu.VMEM((32, _BPW), jnp.float32),
            pltpu.SemaphoreType.DMA,
            pltpu.SemaphoreType.DMA,
        ],
    )
    def k(uidx_hbm, iidx_hbm, fug_r, fig_r, fum_r, fim_r,
          ugt_r, igt_r, umt_r, imt_r,
          ug_o, ig_o, um_o, im_o,
          us, is_, ibu, ibi, ibu2, ibi2, tgb, tmb, ugv, igv, umv, imv,
          semA, semB):
        wid = lax.axis_index("s") * 2 + lax.axis_index("c")
        base = wid * _BPW
        pltpu.sync_copy(uidx_hbm.at[pl.ds(base, _BPW)], us.at[pl.ds(0, _BPW)])
        pltpu.sync_copy(iidx_hbm.at[pl.ds(base, _BPW)], is_.at[pl.ds(0, _BPW)])

        rows16 = lax.iota(jnp.int32, 16)
        m8 = rows16 < 8

        def build(cb, bu, bi):
            for t in range(8):
                uv = jnp.minimum(us[pl.ds(cb + t * 16, 16)], _CUT_U - 1)
                qv = jnp.minimum(is_[pl.ds(cb + t * 16, 16)], _CUT_I - 1)
                ju = (uv >> 7) * 1024 + (uv & 127)
                ji = (qv >> 7) * 1024 + (qv & 127)
                for c in range(8):
                    bu[c, pl.ds(t * 16, 16)] = ju + c * 128
                    bi[c, pl.ds(t * 16, 16)] = ji + c * 128
                for c in range(32):
                    bu[8 + c, pl.ds(t * 16, 16)] = (
                        ju + (c // 8) * _WP_U + (c % 8) * 128)
                    bi[8 + c, pl.ds(t * 16, 16)] = (
                        ji + (c // 8) * _WP_I + (c % 8) * 128)

        def copies(cb, bu, bi, sem):
            out = []
            for c in range(8):
                out.append(pltpu.make_async_copy(
                    fug_r.at[bu.at[c]], ugv.at[c, pl.ds(cb, 128)], sem))
                out.append(pltpu.make_async_copy(
                    fig_r.at[bi.at[c]], igv.at[c, pl.ds(cb, 128)], sem))
            for c in range(32):
                out.append(pltpu.make_async_copy(
                    fum_r.at[bu.at[8 + c]], umv.at[c, pl.ds(cb, 128)], sem))
                out.append(pltpu.make_async_copy(
                    fim_r.at[bi.at[8 + c]], imv.at[c, pl.ds(cb, 128)], sem))
            return out

        def superchunk(g, carry):
            cprev = (g - 1) * 256
            c0 = g * 256

            @pl.when(g > 0)
            def _():
                for cp in copies(cprev, ibu, ibi, semA):
                    cp.wait()

            build(c0, ibu, ibi)
            for cp in copies(c0, ibu, ibi, semA):
                cp.start()

            @pl.when(g > 0)
            def _():
                for cp in copies(cprev + 128, ibu2, ibi2, semB):
                    cp.wait()

            build(c0 + 128, ibu2, ibi2)
            for cp in copies(c0 + 128, ibu2, ibi2, semB):
                cp.start()
            return carry

        nsuper = _BPW // 256
        lax.fori_loop(0, nsuper, superchunk, 0)
        clast = (nsuper - 1) * 256
        for cp in copies(clast, ibu, ibi, semA):
            cp.wait()
        for cp in copies(clast + 128, ibu2, ibi2, semB):
            cp.wait()

        def fixup(gg, carry):
            gb = gg * 16
            uv = us[pl.ds(gb, 16)]
            qv = is_[pl.ds(gb, 16)]

            @pl.when(jnp.max(uv, axis=0) >= _CUT_U)
            def _():
                for e in range(16):
                    r = uv[e]

                    @pl.when(r >= _CUT_U)
                    def _():
                        ju = pl.multiple_of((r >> 7) << 7, 128)
                        pltpu.sync_copy(ugt_r.at[:, pl.ds(ju, 128)], tgb)
                        pltpu.sync_copy(umt_r.at[:, pl.ds(ju, 128)], tmb)
                        k16 = jnp.full((16,), gb + e, jnp.int32)
                        l16 = jnp.full((16,), r & 127, jnp.int32)
                        vg = plsc.load_gather(tgb, [rows16, l16], mask=m8)
                        plsc.store_scatter(ugv, [rows16, k16], vg, mask=m8)
                        for h in range(2):
                            rh = rows16 + (16 * h)
                            vm = plsc.load_gather(tmb, [rh, l16])
                            plsc.store_scatter(umv, [rh, k16], vm)

            @pl.when(jnp.max(qv, axis=0) >= _CUT_I)
            def _():
                for e in range(16):
                    q = qv[e]

                    @pl.when(q >= _CUT_I)
                    def _():
                        ji = pl.multiple_of((q >> 7) << 7, 128)
                        pltpu.sync_copy(igt_r.at[:, pl.ds(ji, 128)], tgb)
                        pltpu.sync_copy(imt_r.at[:, pl.ds(ji, 128)], tmb)
                        k16 = jnp.full((16,), gb + e, jnp.int32)
                        l16 = jnp.full((16,), q & 127, jnp.int32)
                        vg = plsc.load_gather(tgb, [rows16, l16], mask=m8)
                        plsc.store_scatter(igv, [rows16, k16], vg, mask=m8)
                        for h in range(2):
                            rh = rows16 + (16 * h)
                            vm = plsc.load_gather(tmb, [rh, l16])
                            plsc.store_scatter(imv, [rh, k16], vm)
            return carry

        lax.fori_loop(0, _BPW // 16, fixup, 0)

        pltpu.sync_copy(ugv, ug_o.at[:, pl.ds(base, _BPW)])
        pltpu.sync_copy(igv, ig_o.at[:, pl.ds(base, _BPW)])
        pltpu.sync_copy(umv, um_o.at[:, pl.ds(base, _BPW)])
        pltpu.sync_copy(imv, im_o.at[:, pl.ds(base, _BPW)])

    return k(uidx, iidx, fug, fig, fum, fim, ugt, igt, umt, imt)


def _tc_head_t(ugT, igT, umT, imT, W1uT, W1iT, b1c, W2T, b2c, W3T, b3c,
               Wfg, Wfh, bf):
    bb = 2048
    grid = (_BATCH // bb,)

    def body(ug_r, ig_r, um_r, im_r, w1u_r, w1i_r, b1_r, w2_r, b2_r,
             w3_r, b3_r, wfg_r, wfh_r, bf_r, o_r):
        g = ug_r[...] * ig_r[...]
        h = jnp.dot(w1u_r[...], um_r[...], preferred_element_type=jnp.float32)
        h = h + jnp.dot(w1i_r[...], im_r[...], preferred_element_type=jnp.float32)
        h = jnp.maximum(h + b1_r[...], 0.0)
        h = jnp.maximum(
            jnp.dot(w2_r[...], h, preferred_element_type=jnp.float32)
            + b2_r[...], 0.0)
        h = jnp.maximum(
            jnp.dot(w3_r[...], h, preferred_element_type=jnp.float32)
            + b3_r[...], 0.0)
        dn = (((0,), (0,)), ((), ()))
        s = lax.dot_general(g, wfg_r[...], dn,
                            preferred_element_type=jnp.float32)
        s = s + lax.dot_general(h, wfh_r[...], dn,
                                preferred_element_type=jnp.float32)
        s = s + bf_r[...]
        o_r[...] = jax.nn.sigmoid(s)

    batch_spec = lambda d: pl.BlockSpec((d, bb), lambda i: (0, i))
    full_spec = lambda a: pl.BlockSpec(a.shape, lambda i: (0,) * a.ndim)
    return pl.pallas_call(
        body,
        grid=grid,
        in_specs=[
            batch_spec(8), batch_spec(8), batch_spec(32), batch_spec(32),
            full_spec(W1uT), full_spec(W1iT), full_spec(b1c),
            full_spec(W2T), full_spec(b2c), full_spec(W3T), full_spec(b3c),
            full_spec(Wfg), full_spec(Wfh), full_spec(bf),
        ],
        out_specs=pl.BlockSpec((bb, 1), lambda i: (i, 0)),
        out_shape=jax.ShapeDtypeStruct((_BATCH, 1), jnp.float32),
    )(ugT, igT, umT, imT, W1uT, W1iT, b1c, W2T, b2c, W3T, b3c, Wfg, Wfh, bf)


def kernel(user_indices, item_indices, U_gmf, I_gmf, U_mlp, I_mlp,
           W1, b1, W2, b2, W3, b3, Wf, bf):
    uidx = user_indices.astype(jnp.int32)
    iidx = item_indices.astype(jnp.int32)
    ugT, igT, umT, imT = _sc_gather_t(
        uidx, iidx,
        _flat_view(U_gmf, _CUT_U), _flat_view(I_gmf, _CUT_I),
        _flat_view(U_mlp, _CUT_U), _flat_view(I_mlp, _CUT_I),
        U_gmf.T, I_gmf.T, U_mlp.T, I_mlp.T)
    return _tc_head_t(
        ugT, igT, umT, imT,
        W1[:32].T, W1[32:].T, b1.reshape(32, 1),
        W2.T, b2.reshape(16, 1), W3.T, b3.reshape(8, 1),
        Wf[:8], Wf[8:], bf.reshape(1, 1),
    )
